# Initial kernel scaffold; baseline (speedup 1.0000x reference)
import jax
import jax.numpy as jnp
from jax.experimental import pallas as pl


def kernel(inp, padding_mask, regular_tokens_mask):
    raise NotImplementedError("write your pallas kernel here")


# speedup vs baseline: 231.0151x; 231.0151x over previous
import functools
import random

import jax
import jax.numpy as jnp
import numpy as np
from jax import lax
from jax.experimental import pallas as pl
from jax.experimental.pallas import tpu as pltpu
from jax.experimental.pallas import tpu_sc as plsc

_B = 16
_L = 4096
_SPAN = 8
_NCHUNK = _L // 16
_TPAD = _B * _L + 16

_rng = random.Random(0)
_TABLE = np.fromiter((_rng.randrange(1, 2 * _SPAN) for _ in range(_B * _L)),
                     dtype=np.int32, count=_B * _L)


def _iota():
    return lax.iota(jnp.int32, 16)


def _rgather(vec, idx):
    return vec.at[idx].get(mode="promise_in_bounds")


@functools.partial(
    pl.kernel,
    out_type=(jax.ShapeDtypeStruct((_B, _L), jnp.int32),
              jax.ShapeDtypeStruct((_B,), jnp.int32)),
    mesh=plsc.VectorSubcoreMesh(core_axis_name="c", subcore_axis_name="s"),
    scratch_types=[
        pltpu.VMEM((_TPAD,), jnp.int32),
        pltpu.VMEM((_L,), jnp.int32),
        pltpu.VMEM((_L,), jnp.int32),
        pltpu.VMEM((16,), jnp.int32),
    ],
)
def _chunker(rtm_hbm, table_hbm, seg_hbm, cnt_hbm,
             table_v, row_v, seg_v, cnt_v):
    cid = lax.axis_index("c")
    sid = lax.axis_index("s")

    @pl.when((cid == 0) & (sid == 0))
    def _work():
        pltpu.sync_copy(table_hbm, table_v.at[pl.ds(0, _B * _L)])
        lane = _iota()
        lane_m1 = jnp.maximum(lane - 1, 0)
        zeros = lane & 0
        firstlane = 1 - jnp.minimum(lane, 1)

        def row_body(b, carry):
            c0, nxt0, cnt_acc = carry
            pltpu.sync_copy(rtm_hbm.at[b], row_v)

            lv_vec = _rgather(row_v[pl.ds(_L - 16, 16)], zeros + 15)

            def pre_body(i, a):
                vals = row_v[pl.ds(i * 16, 16)]
                pos = lane + i * 16
                diff = jnp.minimum(vals ^ lv_vec, 1)
                return jnp.maximum(a, diff * (pos + 1) - 1)

            pacc = lax.fori_loop(0, _NCHUNK, pre_body, zeros - 1)
            for s in (8, 4, 2, 1):
                pacc = jnp.maximum(pacc, _rgather(pacc, (lane + s) & 15))
            fb_vec = lv_vec * _L + (1 - lv_vec) * jnp.maximum(pacc + 2, 1)

            def chunk_body(i, ch_carry):
                c_s, nxt_s, r_s, prev_last = ch_carry
                base = i * 16
                vals = row_v[pl.ds(base, 16)]
                pos = lane + base
                shifted = _rgather(vals, lane_m1)
                vals0 = vals[0]
                xor0 = jnp.where(i == 0, jnp.int32(1), vals0 ^ prev_last)
                ns_i = (vals ^ shifted) | (firstlane * xor0)
                vld_i = jnp.clip(fb_vec - pos, 0, 1)
                nsvld = ns_i * vld_i
                seg_acc = zeros

                for j in range(16):
                    p = base + j
                    ns_j = nsvld[j] != 0
                    vld_j = vld_i[j] != 0
                    consume = ns_j | (vld_j & (p == nxt_s))
                    d = table_v[pl.ds(c_s, 16)][0]
                    nxt_s = jnp.where(consume, p + d, nxt_s)
                    ci = jnp.where(consume, 1, 0)
                    c_s = c_s + ci
                    r_s = r_s + ci
                    seg_j = jnp.where(vld_j, r_s - 1, -1)
                    eq_j = 1 - jnp.minimum(jnp.abs(lane - j), 1)
                    seg_acc = seg_acc + eq_j * seg_j

                seg_v[pl.ds(base, 16)] = seg_acc
                return (c_s, nxt_s, r_s, vals[15])

            c_end, nxt_end, _, _ = lax.fori_loop(
                0, _NCHUNK, chunk_body, (c0, nxt0, jnp.int32(0), jnp.int32(0)))
            pltpu.sync_copy(seg_v, seg_hbm.at[b])
            eq_b = 1 - jnp.minimum(jnp.abs(lane - b), 1)
            cnt_acc = cnt_acc + eq_b * (c_end - c0)
            return (c_end, nxt_end, cnt_acc)

        init = (jnp.int32(0), jnp.int32(-1), zeros)
        _, _, cnt_final = lax.fori_loop(0, _B, row_body, init)
        cnt_v[pl.ds(0, 16)] = cnt_final
        pltpu.sync_copy(cnt_v, cnt_hbm)


def kernel(inp, padding_mask, regular_tokens_mask):
    del inp, padding_mask
    table = jnp.asarray(_TABLE)
    seg_ids, n_chunks = _chunker(regular_tokens_mask.astype(jnp.int32), table)
    return (seg_ids, n_chunks)


# speedup vs baseline: 261.7345x; 1.1330x over previous
import functools
import random

import jax
import jax.numpy as jnp
import numpy as np
from jax import lax
from jax.experimental import pallas as pl
from jax.experimental.pallas import tpu as pltpu
from jax.experimental.pallas import tpu_sc as plsc

_B = 16
_L = 4096
_SPAN = 8
_NCHUNK = _L // 16
_TPAD = _B * _L + 16

_rng = random.Random(0)
_TABLE = np.fromiter((_rng.randrange(1, 2 * _SPAN) for _ in range(_B * _L)),
                     dtype=np.int32, count=_B * _L)


def _iota():
    return lax.iota(jnp.int32, 16)


def _rgather(vec, idx):
    return vec.at[idx].get(mode="promise_in_bounds")


@functools.partial(
    pl.kernel,
    out_type=(jax.ShapeDtypeStruct((_B, _L), jnp.int32),
              jax.ShapeDtypeStruct((_B,), jnp.int32)),
    mesh=plsc.VectorSubcoreMesh(core_axis_name="c", subcore_axis_name="s"),
    scratch_types=[
        pltpu.VMEM((_TPAD,), jnp.int32),
        pltpu.VMEM((_L,), jnp.int32),
        pltpu.VMEM((_L,), jnp.int32),
        pltpu.VMEM((16,), jnp.int32),
    ],
)
def _chunker(rtm_hbm, table_hbm, seg_hbm, cnt_hbm,
             table_v, row_v, seg_v, cnt_v):
    cid = lax.axis_index("c")
    sid = lax.axis_index("s")

    @pl.when((cid == 0) & (sid == 0))
    def _work():
        pltpu.sync_copy(table_hbm, table_v.at[pl.ds(0, _B * _L)])
        lane = _iota()
        lane_m1 = jnp.maximum(lane - 1, 0)
        lane_p1 = jnp.minimum(lane + 1, 15)
        zeros = lane & 0
        firstlane = 1 - jnp.minimum(lane, 1)

        def row_body(b, carry):
            c0, nxt0, cnt_acc = carry
            pltpu.sync_copy(rtm_hbm.at[b], row_v)

            lv_vec = _rgather(row_v[pl.ds(_L - 16, 16)], zeros + 15)

            def pre_body(i, a):
                vals = row_v[pl.ds(i * 16, 16)]
                pos = lane + i * 16
                diff = jnp.minimum(vals ^ lv_vec, 1)
                return jnp.maximum(a, diff * (pos + 1) - 1)

            pacc = lax.fori_loop(0, _NCHUNK, pre_body, zeros - 1)
            for s in (8, 4, 2, 1):
                pacc = jnp.maximum(pacc, _rgather(pacc, (lane + s) & 15))
            fb_vec = lv_vec * _L + (1 - lv_vec) * jnp.maximum(pacc + 2, 1)

            def chunk_body(i, ch_carry):
                c_s, nxt_s, prev_last = ch_carry
                base = i * 16
                c_chunk0 = c_s
                vals = row_v[pl.ds(base, 16)]
                pos = lane + base
                shifted = _rgather(vals, lane_m1)
                vals0 = vals[0]
                xor0 = jnp.where(i == 0, jnp.int32(1), vals0 ^ prev_last)
                ns_i = (vals ^ shifted) | (firstlane * xor0)
                vld_i = jnp.clip(fb_vec - pos, 0, 1)
                nsvld = ns_i * vld_i
                tw = table_v[pl.ds(c_s, 16)]
                cbits = jnp.int32(0)

                for j in range(16):
                    p = base + j
                    d = tw[0]
                    pd = p + d
                    ns_j = nsvld[j] != 0
                    vld_j = vld_i[j] != 0
                    consume = ns_j | (vld_j & (p == nxt_s))
                    nxt_s = jnp.where(consume, pd, nxt_s)
                    ci = jnp.where(consume, 1, 0)
                    c_s = c_s + ci
                    cbits = cbits + (ci << j)
                    tw_shift = _rgather(tw, lane_p1)
                    tw = jnp.where(consume, tw_shift, tw)

                bits_vec = jnp.right_shift(zeros + cbits, lane) & 1
                csum = bits_vec
                for s in (1, 2, 4, 8):
                    ind = jnp.clip(lane - s + 1, 0, 1)
                    csum = csum + _rgather(csum, jnp.maximum(lane - s, 0)) * ind
                c_incl = c_chunk0 + csum
                seg_v[pl.ds(base, 16)] = vld_i * (c_incl - c0) - 1
                return (c_s, nxt_s, vals[15])

            c_end, nxt_end, _ = lax.fori_loop(
                0, _NCHUNK, chunk_body, (c0, nxt0, jnp.int32(0)))
            pltpu.sync_copy(seg_v, seg_hbm.at[b])
            eq_b = 1 - jnp.minimum(jnp.abs(lane - b), 1)
            cnt_acc = cnt_acc + eq_b * (c_end - c0)
            return (c_end, nxt_end, cnt_acc)

        init = (jnp.int32(0), jnp.int32(-1), zeros)
        _, _, cnt_final = lax.fori_loop(0, _B, row_body, init)
        cnt_v[pl.ds(0, 16)] = cnt_final
        pltpu.sync_copy(cnt_v, cnt_hbm)


def kernel(inp, padding_mask, regular_tokens_mask):
    del inp, padding_mask
    table = jnp.asarray(_TABLE)
    seg_ids, n_chunks = _chunker(regular_tokens_mask.astype(jnp.int32), table)
    return (seg_ids, n_chunks)
